# Initial kernel scaffold; baseline (speedup 1.0000x reference)
import jax
import jax.numpy as jnp
from jax.experimental import pallas as pl


def kernel(x, batch, W1, b1, W2, b2):
    raise NotImplementedError("write your pallas kernel here")


# speedup vs baseline: 7.0073x; 7.0073x over previous
import functools

import jax
import jax.numpy as jnp
from jax.experimental import pallas as pl
from jax.experimental.pallas import tpu as pltpu

N = 100000
D = 128
NUM_GRAPHS = 64
BLOCK = 2000
NUM_BLOCKS = N // BLOCK


def _fused_kernel(x_ref, batch_ref, w1_ref, b1_ref, w2_ref, b2_ref,
                  out_ref, acc_ref, m_ref, s_ref):
    i = pl.program_id(0)

    @pl.when(i == 0)
    def _init():
        acc_ref[...] = jnp.zeros_like(acc_ref)
        m_ref[0, 0] = -jnp.inf
        s_ref[0, 0] = 0.0

    xb = x_ref[...]
    h = jnp.tanh(
        jnp.dot(xb, w1_ref[...], preferred_element_type=jnp.float32)
        + b1_ref[...]
    )
    logits = jnp.sum(h * w2_ref[...], axis=1, keepdims=True)
    logits = logits + b2_ref[0, 0]

    m_old = m_ref[0, 0]
    m_new = jnp.maximum(m_old, jnp.max(logits))
    corr = jnp.exp(m_old - m_new)
    p = jnp.exp(logits - m_new)
    s_ref[0, 0] = s_ref[0, 0] * corr + jnp.sum(p)
    m_ref[0, 0] = m_new

    bb = batch_ref[0, 0, :]
    onehot = (jax.lax.broadcasted_iota(jnp.int32, (NUM_GRAPHS, BLOCK), 0)
              == bb[None, :]).astype(jnp.float32)
    seg = jnp.dot(onehot, xb * p, preferred_element_type=jnp.float32)
    acc_ref[...] = acc_ref[...] * corr + seg

    @pl.when(i == NUM_BLOCKS - 1)
    def _fin():
        out_ref[...] = acc_ref[...] / s_ref[0, 0]


@jax.jit
def kernel(x, batch, W1, b1, W2, b2):
    batch3 = batch.astype(jnp.int32).reshape(NUM_BLOCKS, 1, BLOCK)
    b1r = b1.reshape(1, D)
    w2r = W2.reshape(1, D)
    b2r = b2.reshape(1, 1)
    out = pl.pallas_call(
        _fused_kernel,
        grid=(NUM_BLOCKS,),
        in_specs=[
            pl.BlockSpec((BLOCK, D), lambda i: (i, 0)),
            pl.BlockSpec((1, 1, BLOCK), lambda i: (i, 0, 0)),
            pl.BlockSpec((D, D), lambda i: (0, 0)),
            pl.BlockSpec((1, D), lambda i: (0, 0)),
            pl.BlockSpec((1, D), lambda i: (0, 0)),
            pl.BlockSpec((1, 1), lambda i: (0, 0)),
        ],
        out_specs=pl.BlockSpec((NUM_GRAPHS, D), lambda i: (0, 0)),
        out_shape=jax.ShapeDtypeStruct((NUM_GRAPHS, D), jnp.float32),
        scratch_shapes=[
            pltpu.VMEM((NUM_GRAPHS, D), jnp.float32),
            pltpu.SMEM((1, 1), jnp.float32),
            pltpu.SMEM((1, 1), jnp.float32),
        ],
    )(x, batch3, W1, b1r, w2r, b2r)
    return out


# speedup vs baseline: 7.3709x; 1.0519x over previous
import functools

import jax
import jax.numpy as jnp
from jax.experimental import pallas as pl
from jax.experimental.pallas import tpu as pltpu

N = 100000
D = 128
NUM_GRAPHS = 64
BLOCK = 2000
NUM_BLOCKS = N // BLOCK


def _fused_kernel(x_ref, batch_ref, w1_ref, b1_ref, w2_ref, b2_ref,
                  out_ref, acc_ref, m_ref, s_ref):
    i = pl.program_id(0)

    @pl.when(i == 0)
    def _init():
        acc_ref[...] = jnp.zeros_like(acc_ref)
        m_ref[0, 0] = -jnp.inf
        s_ref[0, 0] = 0.0

    xb = x_ref[...]
    h = jnp.tanh(
        jnp.dot(xb, w1_ref[...], preferred_element_type=jnp.float32)
        + b1_ref[...]
    )
    logits = jax.lax.dot_general(
        w2_ref[...], h, (((1,), (1,)), ((), ())),
        preferred_element_type=jnp.float32,
    ) + b2_ref[0, 0]

    m_old = m_ref[0, 0]
    m_new = jnp.maximum(m_old, jnp.max(logits))
    corr = jnp.exp(m_old - m_new)
    p = jnp.exp(logits - m_new)
    s_ref[0, 0] = s_ref[0, 0] * corr + jnp.sum(p)
    m_ref[0, 0] = m_new

    bb = batch_ref[0]
    onehot = (jax.lax.broadcasted_iota(jnp.int32, (NUM_GRAPHS, BLOCK), 0)
              == bb).astype(jnp.float32) * p
    seg = jnp.dot(onehot, xb, preferred_element_type=jnp.float32)
    acc_ref[...] = acc_ref[...] * corr + seg

    @pl.when(i == NUM_BLOCKS - 1)
    def _fin():
        out_ref[...] = acc_ref[...] / s_ref[0, 0]


@jax.jit
def kernel(x, batch, W1, b1, W2, b2):
    batch3 = batch.astype(jnp.int32).reshape(NUM_BLOCKS, 1, BLOCK)
    b1r = b1.reshape(1, D)
    w2r = W2.reshape(1, D)
    b2r = b2.reshape(1, 1)
    out = pl.pallas_call(
        _fused_kernel,
        grid=(NUM_BLOCKS,),
        in_specs=[
            pl.BlockSpec((BLOCK, D), lambda i: (i, 0)),
            pl.BlockSpec((1, 1, BLOCK), lambda i: (i, 0, 0)),
            pl.BlockSpec((D, D), lambda i: (0, 0)),
            pl.BlockSpec((1, D), lambda i: (0, 0)),
            pl.BlockSpec((1, D), lambda i: (0, 0)),
            pl.BlockSpec((1, 1), lambda i: (0, 0)),
        ],
        out_specs=pl.BlockSpec((NUM_GRAPHS, D), lambda i: (0, 0)),
        out_shape=jax.ShapeDtypeStruct((NUM_GRAPHS, D), jnp.float32),
        scratch_shapes=[
            pltpu.VMEM((NUM_GRAPHS, D), jnp.float32),
            pltpu.SMEM((1, 1), jnp.float32),
            pltpu.SMEM((1, 1), jnp.float32),
        ],
    )(x, batch3, W1, b1r, w2r, b2r)
    return out


# speedup vs baseline: 10.8866x; 1.4770x over previous
import functools

import jax
import jax.numpy as jnp
from jax.experimental import pallas as pl
from jax.experimental.pallas import tpu as pltpu

N = 100000
D = 128
NUM_GRAPHS = 64
BLOCK = 4000
NUM_BLOCKS = N // BLOCK


def _fused_kernel(x_ref, batch_ref, w1_ref, b1_ref, w2_ref, b2_ref,
                  out_ref, acc_ref, m_ref, s_ref):
    i = pl.program_id(0)

    @pl.when(i == 0)
    def _init():
        acc_ref[...] = jnp.zeros_like(acc_ref)
        m_ref[0, 0] = -jnp.inf
        s_ref[0, 0] = 0.0

    xb = x_ref[...]
    h = jnp.tanh(
        jnp.dot(xb, w1_ref[...], preferred_element_type=jnp.float32)
        + b1_ref[...]
    )
    logits = jax.lax.dot_general(
        w2_ref[...], h, (((1,), (1,)), ((), ())),
        preferred_element_type=jnp.float32,
    ) + b2_ref[0, 0]

    m_old = m_ref[0, 0]
    m_new = jnp.maximum(m_old, jnp.max(logits))
    corr = jnp.exp(m_old - m_new)
    p = jnp.exp(logits - m_new)
    s_ref[0, 0] = s_ref[0, 0] * corr + jnp.sum(p)
    m_ref[0, 0] = m_new

    bb = batch_ref[0]
    onehot = (jax.lax.broadcasted_iota(jnp.int32, (NUM_GRAPHS, BLOCK), 0)
              == bb).astype(jnp.float32) * p
    seg = jnp.dot(onehot, xb, preferred_element_type=jnp.float32)
    acc_ref[...] = acc_ref[...] * corr + seg

    @pl.when(i == NUM_BLOCKS - 1)
    def _fin():
        out_ref[...] = acc_ref[...] / s_ref[0, 0]


@jax.jit
def kernel(x, batch, W1, b1, W2, b2):
    batch3 = batch.astype(jnp.int32).reshape(NUM_BLOCKS, 1, BLOCK)
    b1r = b1.reshape(1, D)
    w2r = W2.reshape(1, D)
    b2r = b2.reshape(1, 1)
    out = pl.pallas_call(
        _fused_kernel,
        grid=(NUM_BLOCKS,),
        in_specs=[
            pl.BlockSpec((BLOCK, D), lambda i: (i, 0)),
            pl.BlockSpec((1, 1, BLOCK), lambda i: (i, 0, 0)),
            pl.BlockSpec((D, D), lambda i: (0, 0)),
            pl.BlockSpec((1, D), lambda i: (0, 0)),
            pl.BlockSpec((1, D), lambda i: (0, 0)),
            pl.BlockSpec((1, 1), lambda i: (0, 0)),
        ],
        out_specs=pl.BlockSpec((NUM_GRAPHS, D), lambda i: (0, 0)),
        out_shape=jax.ShapeDtypeStruct((NUM_GRAPHS, D), jnp.float32),
        scratch_shapes=[
            pltpu.VMEM((NUM_GRAPHS, D), jnp.float32),
            pltpu.SMEM((1, 1), jnp.float32),
            pltpu.SMEM((1, 1), jnp.float32),
        ],
    )(x, batch3, W1, b1r, w2r, b2r)
    return out


# speedup vs baseline: 13.8975x; 1.2766x over previous
import functools

import jax
import jax.numpy as jnp
from jax.experimental import pallas as pl
from jax.experimental.pallas import tpu as pltpu

N = 100000
D = 128
NUM_GRAPHS = 64
BLOCK = 10000
NUM_BLOCKS = N // BLOCK


def _fused_kernel(x_ref, batch_ref, w1_ref, b1_ref, w2_ref, b2_ref,
                  out_ref, acc_ref, m_ref, s_ref):
    i = pl.program_id(0)

    @pl.when(i == 0)
    def _init():
        acc_ref[...] = jnp.zeros_like(acc_ref)
        m_ref[0, 0] = -jnp.inf
        s_ref[0, 0] = 0.0

    xb = x_ref[...]
    h = jnp.tanh(
        jnp.dot(xb, w1_ref[...], preferred_element_type=jnp.float32)
        + b1_ref[...]
    )
    logits = jax.lax.dot_general(
        w2_ref[...], h, (((1,), (1,)), ((), ())),
        preferred_element_type=jnp.float32,
    ) + b2_ref[0, 0]

    m_old = m_ref[0, 0]
    m_new = jnp.maximum(m_old, jnp.max(logits))
    corr = jnp.exp(m_old - m_new)
    p = jnp.exp(logits - m_new)
    s_ref[0, 0] = s_ref[0, 0] * corr + jnp.sum(p)
    m_ref[0, 0] = m_new

    bb = batch_ref[0]
    onehot = (jax.lax.broadcasted_iota(jnp.int32, (NUM_GRAPHS, BLOCK), 0)
              == bb).astype(jnp.float32) * p
    seg = jnp.dot(onehot, xb, preferred_element_type=jnp.float32)
    acc_ref[...] = acc_ref[...] * corr + seg

    @pl.when(i == NUM_BLOCKS - 1)
    def _fin():
        out_ref[...] = acc_ref[...] / s_ref[0, 0]


@jax.jit
def kernel(x, batch, W1, b1, W2, b2):
    batch3 = batch.astype(jnp.int32).reshape(NUM_BLOCKS, 1, BLOCK)
    b1r = b1.reshape(1, D)
    w2r = W2.reshape(1, D)
    b2r = b2.reshape(1, 1)
    out = pl.pallas_call(
        _fused_kernel,
        grid=(NUM_BLOCKS,),
        in_specs=[
            pl.BlockSpec((BLOCK, D), lambda i: (i, 0)),
            pl.BlockSpec((1, 1, BLOCK), lambda i: (i, 0, 0)),
            pl.BlockSpec((D, D), lambda i: (0, 0)),
            pl.BlockSpec((1, D), lambda i: (0, 0)),
            pl.BlockSpec((1, D), lambda i: (0, 0)),
            pl.BlockSpec((1, 1), lambda i: (0, 0)),
        ],
        out_specs=pl.BlockSpec((NUM_GRAPHS, D), lambda i: (0, 0)),
        out_shape=jax.ShapeDtypeStruct((NUM_GRAPHS, D), jnp.float32),
        scratch_shapes=[
            pltpu.VMEM((NUM_GRAPHS, D), jnp.float32),
            pltpu.SMEM((1, 1), jnp.float32),
            pltpu.SMEM((1, 1), jnp.float32),
        ],
    )(x, batch3, W1, b1r, w2r, b2r)
    return out


# speedup vs baseline: 14.1878x; 1.0209x over previous
import functools

import jax
import jax.numpy as jnp
from jax.experimental import pallas as pl
from jax.experimental.pallas import tpu as pltpu

N = 100000
D = 128
NUM_GRAPHS = 64
BLOCK = 20000
NUM_BLOCKS = N // BLOCK


def _fused_kernel(x_ref, batch_ref, w1_ref, b1_ref, w2_ref, b2_ref,
                  out_ref, acc_ref, m_ref, s_ref):
    i = pl.program_id(0)

    @pl.when(i == 0)
    def _init():
        acc_ref[...] = jnp.zeros_like(acc_ref)
        m_ref[0, 0] = -jnp.inf
        s_ref[0, 0] = 0.0

    xb = x_ref[...]
    h = jnp.tanh(
        jnp.dot(xb, w1_ref[...], preferred_element_type=jnp.float32)
        + b1_ref[...]
    )
    logits = jax.lax.dot_general(
        w2_ref[...], h, (((1,), (1,)), ((), ())),
        preferred_element_type=jnp.float32,
    ) + b2_ref[0, 0]

    m_old = m_ref[0, 0]
    m_new = jnp.maximum(m_old, jnp.max(logits))
    corr = jnp.exp(m_old - m_new)
    p = jnp.exp(logits - m_new)
    s_ref[0, 0] = s_ref[0, 0] * corr + jnp.sum(p)
    m_ref[0, 0] = m_new

    bb = batch_ref[0]
    onehot = (jax.lax.broadcasted_iota(jnp.int32, (NUM_GRAPHS, BLOCK), 0)
              == bb).astype(jnp.float32) * p
    seg = jnp.dot(onehot, xb, preferred_element_type=jnp.float32)
    acc_ref[...] = acc_ref[...] * corr + seg

    @pl.when(i == NUM_BLOCKS - 1)
    def _fin():
        out_ref[...] = acc_ref[...] / s_ref[0, 0]


@jax.jit
def kernel(x, batch, W1, b1, W2, b2):
    batch3 = batch.astype(jnp.int32).reshape(NUM_BLOCKS, 1, BLOCK)
    b1r = b1.reshape(1, D)
    w2r = W2.reshape(1, D)
    b2r = b2.reshape(1, 1)
    out = pl.pallas_call(
        _fused_kernel,
        grid=(NUM_BLOCKS,),
        in_specs=[
            pl.BlockSpec((BLOCK, D), lambda i: (i, 0)),
            pl.BlockSpec((1, 1, BLOCK), lambda i: (i, 0, 0)),
            pl.BlockSpec((D, D), lambda i: (0, 0)),
            pl.BlockSpec((1, D), lambda i: (0, 0)),
            pl.BlockSpec((1, D), lambda i: (0, 0)),
            pl.BlockSpec((1, 1), lambda i: (0, 0)),
        ],
        out_specs=pl.BlockSpec((NUM_GRAPHS, D), lambda i: (0, 0)),
        out_shape=jax.ShapeDtypeStruct((NUM_GRAPHS, D), jnp.float32),
        scratch_shapes=[
            pltpu.VMEM((NUM_GRAPHS, D), jnp.float32),
            pltpu.SMEM((1, 1), jnp.float32),
            pltpu.SMEM((1, 1), jnp.float32),
        ],
    )(x, batch3, W1, b1r, w2r, b2r)
    return out


# speedup vs baseline: 16.0534x; 1.1315x over previous
import functools

import jax
import jax.numpy as jnp
from jax.experimental import pallas as pl
from jax.experimental.pallas import tpu as pltpu

N = 100000
D = 128
NUM_GRAPHS = 64
BLOCK = 20000
NUM_BLOCKS = N // BLOCK


def _fused_kernel(x_ref, batch_ref, w1_ref, b1_ref, w2_ref, b2_ref,
                  out_ref, acc_ref, m_ref, s_ref):
    i = pl.program_id(0)

    @pl.when(i == 0)
    def _init():
        acc_ref[...] = jnp.zeros_like(acc_ref)
        m_ref[0, 0] = -jnp.inf
        s_ref[0, 0] = 0.0

    xb = x_ref[...]
    h = jnp.tanh(
        jnp.dot(xb.astype(jnp.bfloat16), w1_ref[...].astype(jnp.bfloat16),
                preferred_element_type=jnp.float32)
        + b1_ref[...]
    )
    logits = jax.lax.dot_general(
        w2_ref[...], h, (((1,), (1,)), ((), ())),
        preferred_element_type=jnp.float32,
    ) + b2_ref[0, 0]

    m_old = m_ref[0, 0]
    m_new = jnp.maximum(m_old, jnp.max(logits))
    corr = jnp.exp(m_old - m_new)
    p = jnp.exp(logits - m_new)
    s_ref[0, 0] = s_ref[0, 0] * corr + jnp.sum(p)
    m_ref[0, 0] = m_new

    bb = batch_ref[0]
    onehot = jnp.where(
        jax.lax.broadcasted_iota(jnp.int32, (NUM_GRAPHS, BLOCK), 0) == bb,
        p, 0.0)
    seg = jnp.dot(onehot, xb, preferred_element_type=jnp.float32)
    acc_ref[...] = acc_ref[...] * corr + seg

    @pl.when(i == NUM_BLOCKS - 1)
    def _fin():
        out_ref[...] = acc_ref[...] / s_ref[0, 0]


@jax.jit
def kernel(x, batch, W1, b1, W2, b2):
    batch3 = batch.astype(jnp.int32).reshape(NUM_BLOCKS, 1, BLOCK)
    b1r = b1.reshape(1, D)
    w2r = W2.reshape(1, D)
    b2r = b2.reshape(1, 1)
    out = pl.pallas_call(
        _fused_kernel,
        grid=(NUM_BLOCKS,),
        in_specs=[
            pl.BlockSpec((BLOCK, D), lambda i: (i, 0)),
            pl.BlockSpec((1, 1, BLOCK), lambda i: (i, 0, 0)),
            pl.BlockSpec((D, D), lambda i: (0, 0)),
            pl.BlockSpec((1, D), lambda i: (0, 0)),
            pl.BlockSpec((1, D), lambda i: (0, 0)),
            pl.BlockSpec((1, 1), lambda i: (0, 0)),
        ],
        out_specs=pl.BlockSpec((NUM_GRAPHS, D), lambda i: (0, 0)),
        out_shape=jax.ShapeDtypeStruct((NUM_GRAPHS, D), jnp.float32),
        scratch_shapes=[
            pltpu.VMEM((NUM_GRAPHS, D), jnp.float32),
            pltpu.SMEM((1, 1), jnp.float32),
            pltpu.SMEM((1, 1), jnp.float32),
        ],
    )(x, batch3, W1, b1r, w2r, b2r)
    return out
